# Initial kernel scaffold; baseline (speedup 1.0000x reference)
import jax
import jax.numpy as jnp
from jax.experimental import pallas as pl


def kernel(x, edge_index, batch, lin_W, lin_b, W1, b1, W2, b2, W3, b3, emb_W, emb_b, pred_W, pred_b):
    raise NotImplementedError("write your pallas kernel here")


# speedup vs baseline: 10.3072x; 10.3072x over previous
import functools

import jax
import jax.numpy as jnp
from jax import lax
from jax.experimental import pallas as pl
from jax.experimental.pallas import tpu as pltpu
from jax.experimental.pallas import tpu_sc as plsc

N = 10000
E = 320000
D_IN = 128
H = 64
G = 64

NC = 2
NS = 16
NW = NC * NS
CH = 128
EP_TILE = 10240
NCHUNK = EP_TILE // CH
E_PAD = NW * EP_TILE
ACC_ROWS = 10240
DEG_ROWS = 10240

_F32 = jnp.float32

_SC_MESH = plsc.VectorSubcoreMesh(core_axis_name="c", subcore_axis_name="s")


@functools.partial(
    pl.kernel,
    out_type=jax.ShapeDtypeStruct((NC, DEG_ROWS), _F32),
    mesh=_SC_MESH,
    scratch_types=[
        pltpu.VMEM((CH,), jnp.int32),
        pltpu.VMEM((CH,), _F32),
        pltpu.VMEM_SHARED((DEG_ROWS,), _F32),
    ],
)
def _sc_degree(dst_hbm, zeros1_hbm, out_hbm, idx_v, ones_v, deg_sh):
    c = lax.axis_index("c")
    s = lax.axis_index("s")
    rows_per_tile = DEG_ROWS // NS
    for i in range(CH // 16):
        ones_v[pl.ds(i * 16, 16)] = jnp.full((16,), 1.0, _F32)
    pltpu.sync_copy(zeros1_hbm.at[pl.ds(s * rows_per_tile, rows_per_tile)],
                    deg_sh.at[pl.ds(s * rows_per_tile, rows_per_tile)])
    plsc.subcore_barrier()
    base = (c * NS + s) * EP_TILE

    def step(i, carry):
        off = base + i * CH
        pltpu.sync_copy(dst_hbm.at[pl.ds(off, CH)], idx_v)
        pltpu.sync_copy(ones_v, deg_sh.at[idx_v], add=True)
        return carry

    lax.fori_loop(0, NCHUNK, step, 0)
    plsc.subcore_barrier()
    pltpu.sync_copy(deg_sh.at[pl.ds(s * rows_per_tile, rows_per_tile)],
                    out_hbm.at[c, pl.ds(s * rows_per_tile, rows_per_tile)])


@functools.partial(
    pl.kernel,
    out_type=jax.ShapeDtypeStruct((NC, ACC_ROWS, H), _F32),
    mesh=_SC_MESH,
    scratch_types=[
        pltpu.VMEM((CH,), jnp.int32),
        pltpu.VMEM((CH,), jnp.int32),
        pltpu.VMEM((CH, H), _F32),
        pltpu.VMEM_SHARED((ACC_ROWS, H), _F32),
        pltpu.SemaphoreType.DMA,
    ],
    compiler_params=pltpu.CompilerParams(use_tc_tiling_on_sc=False),
)
def _sc_aggregate(ht_hbm, src_hbm, dst_hbm, zeros2_hbm, out_hbm,
                  idx_s, idx_d, rows_v, acc_sh, sem):
    c = lax.axis_index("c")
    s = lax.axis_index("s")
    rows_per_tile = ACC_ROWS // NS
    pltpu.sync_copy(zeros2_hbm.at[pl.ds(s * rows_per_tile, rows_per_tile)],
                    acc_sh.at[pl.ds(s * rows_per_tile, rows_per_tile)])
    plsc.subcore_barrier()
    base = (c * NS + s) * EP_TILE

    def step(i, carry):
        off = base + i * CH
        pltpu.sync_copy(src_hbm.at[pl.ds(off, CH)], idx_s)
        pltpu.sync_copy(dst_hbm.at[pl.ds(off, CH)], idx_d)
        pltpu.async_copy(ht_hbm.at[idx_s], rows_v, sem).wait()
        pltpu.sync_copy(rows_v, acc_sh.at[idx_d], add=True)
        return carry

    lax.fori_loop(0, NCHUNK, step, 0)
    plsc.subcore_barrier()
    pltpu.sync_copy(acc_sh.at[pl.ds(s * rows_per_tile, rows_per_tile)],
                    out_hbm.at[c].at[pl.ds(s * rows_per_tile, rows_per_tile)])


def _dis(degp_ref):
    deg = degp_ref[0, :N] + degp_ref[1, :N] + 1.0
    return lax.rsqrt(deg)


def _tc_pre_body(x_ref, lin_W_ref, lin_b_ref, W1_ref, degp_ref, ht_ref):
    dis = _dis(degp_ref)
    h0 = jnp.dot(x_ref[...], lin_W_ref[...], preferred_element_type=_F32)
    h0 = h0 + lin_b_ref[...][None, :]
    ht_ref[...] = dis[:, None] * jnp.dot(h0, W1_ref[...],
                                         preferred_element_type=_F32)


def _tc_mid_body(p_ref, ht_ref, degp_ref, b_ref, Wn_ref, out_ref):
    dis = _dis(degp_ref)
    acc = p_ref[0, :N, :] + p_ref[1, :N, :] + ht_ref[...]
    h = jnp.maximum(dis[:, None] * acc + b_ref[...][None, :], 0.0)
    out_ref[...] = dis[:, None] * jnp.dot(h, Wn_ref[...],
                                          preferred_element_type=_F32)


def _tc_post_body(p_ref, ht_ref, degp_ref, b3_ref, emb_W_ref, emb_b_ref,
                  batch_ref, pred_W_ref, pred_b_ref, out_ref):
    dis = _dis(degp_ref)
    acc = p_ref[0, :N, :] + p_ref[1, :N, :] + ht_ref[...]
    h = jnp.maximum(dis[:, None] * acc + b3_ref[...][None, :], 0.0)
    emb = jnp.dot(h, emb_W_ref[...], preferred_element_type=_F32)
    emb = jnp.maximum(emb + emb_b_ref[...][None, :], 0.0)
    gid = lax.broadcasted_iota(jnp.int32, (G, N), 0)
    onehot_t = (batch_ref[...][None, :] == gid).astype(_F32)
    pooled = jnp.dot(onehot_t, emb, preferred_element_type=_F32)
    w = pred_W_ref[...][:, 0][None, :]
    out_ref[...] = jnp.sum(pooled * w, axis=1) + pred_b_ref[0]


def _tc_call(body, out_shape, *args):
    return pl.pallas_call(body, out_shape=out_shape)(*args)


def kernel(x, edge_index, batch, lin_W, lin_b, W1, b1, W2, b2, W3, b3,
           emb_W, emb_b, pred_W, pred_b):
    src = edge_index[0]
    dst = edge_index[1]
    pad = E_PAD - E
    src_p = jnp.concatenate([src, jnp.zeros((pad,), jnp.int32)])
    dst_p = jnp.concatenate([dst, jnp.full((pad,), N, jnp.int32)])
    zeros1 = jnp.zeros((DEG_ROWS,), _F32)
    zeros2 = jnp.zeros((ACC_ROWS, H), _F32)

    degp = _sc_degree(dst_p, zeros1)

    ht1 = _tc_call(_tc_pre_body, jax.ShapeDtypeStruct((N, H), _F32),
                   x, lin_W, lin_b, W1, degp)
    p1 = _sc_aggregate(ht1, src_p, dst_p, zeros2)
    ht2 = _tc_call(_tc_mid_body, jax.ShapeDtypeStruct((N, H), _F32),
                   p1, ht1, degp, b1, W2)
    p2 = _sc_aggregate(ht2, src_p, dst_p, zeros2)
    ht3 = _tc_call(_tc_mid_body, jax.ShapeDtypeStruct((N, H), _F32),
                   p2, ht2, degp, b2, W3)
    p3 = _sc_aggregate(ht3, src_p, dst_p, zeros2)
    out = _tc_call(_tc_post_body, jax.ShapeDtypeStruct((G,), _F32),
                   p3, ht3, degp, b3, emb_W, emb_b, batch, pred_W, pred_b)
    return out


# speedup vs baseline: 14.5258x; 1.4093x over previous
import functools

import jax
import jax.numpy as jnp
from jax import lax
from jax.experimental import pallas as pl
from jax.experimental.pallas import tpu as pltpu
from jax.experimental.pallas import tpu_sc as plsc

N = 10000
E = 320000
D_IN = 128
H = 64
G = 64

NC = 2
NS = 16
NW = NC * NS
CH = 128
EP_TILE = 10240
NCHUNK = EP_TILE // CH
E_PAD = NW * EP_TILE
ACC_ROWS = 10240
DEG_ROWS = 10240

_F32 = jnp.float32

_SC_MESH = plsc.VectorSubcoreMesh(core_axis_name="c", subcore_axis_name="s")


NBUF = 4
NGROUP = NCHUNK // NBUF


@functools.partial(
    pl.kernel,
    out_type=jax.ShapeDtypeStruct((NC, DEG_ROWS), _F32),
    mesh=_SC_MESH,
    scratch_types=[
        pltpu.VMEM((NCHUNK, CH), jnp.int32),
        pltpu.VMEM((CH,), _F32),
        pltpu.VMEM_SHARED((DEG_ROWS,), _F32),
        pltpu.SemaphoreType.DMA,
    ],
)
def _sc_degree(dst_hbm, zeros1_hbm, out_hbm, idx_v, ones_v, deg_sh, sem):
    c = lax.axis_index("c")
    s = lax.axis_index("s")
    wid = c * NS + s
    rows_per_tile = DEG_ROWS // NS
    for i in range(CH // 16):
        ones_v[pl.ds(i * 16, 16)] = jnp.full((16,), 1.0, _F32)
    pltpu.sync_copy(dst_hbm.at[wid], idx_v)
    pltpu.sync_copy(zeros1_hbm.at[pl.ds(s * rows_per_tile, rows_per_tile)],
                    deg_sh.at[pl.ds(s * rows_per_tile, rows_per_tile)])
    plsc.subcore_barrier()

    def issue(i, carry):
        pltpu.async_copy(ones_v, deg_sh.at[idx_v.at[i]], sem, add=True)
        return carry

    def drain(i, carry):
        pltpu.make_async_copy(ones_v, deg_sh.at[idx_v.at[0]], sem).wait()
        return carry

    lax.fori_loop(0, NCHUNK, issue, 0)
    lax.fori_loop(0, NCHUNK, drain, 0)
    plsc.subcore_barrier()
    pltpu.sync_copy(deg_sh.at[pl.ds(s * rows_per_tile, rows_per_tile)],
                    out_hbm.at[c, pl.ds(s * rows_per_tile, rows_per_tile)])


@functools.partial(
    pl.kernel,
    out_type=jax.ShapeDtypeStruct((NC, ACC_ROWS, H), _F32),
    mesh=_SC_MESH,
    scratch_types=[
        pltpu.VMEM((NCHUNK, CH), jnp.int32),
        pltpu.VMEM((NCHUNK, CH), jnp.int32),
        pltpu.VMEM((NBUF, CH, H), _F32),
        pltpu.VMEM_SHARED((ACC_ROWS, H), _F32),
        pltpu.SemaphoreType.DMA((NBUF,)),
        pltpu.SemaphoreType.DMA((NBUF,)),
    ],
    compiler_params=pltpu.CompilerParams(use_tc_tiling_on_sc=False),
)
def _sc_aggregate(ht_hbm, src_hbm, dst_hbm, zeros2_hbm, out_hbm,
                  idx_s, idx_d, rows_v, acc_sh, sem_g, sem_s):
    c = lax.axis_index("c")
    s = lax.axis_index("s")
    wid = c * NS + s
    rows_per_tile = ACC_ROWS // NS
    pltpu.sync_copy(src_hbm.at[wid], idx_s)
    pltpu.sync_copy(dst_hbm.at[wid], idx_d)
    pltpu.sync_copy(zeros2_hbm.at[pl.ds(s * rows_per_tile, rows_per_tile)],
                    acc_sh.at[pl.ds(s * rows_per_tile, rows_per_tile)])
    plsc.subcore_barrier()

    def start_g(b, chunk):
        pltpu.async_copy(ht_hbm.at[idx_s.at[chunk]], rows_v.at[b], sem_g.at[b])

    def wait_g(b):
        pltpu.make_async_copy(ht_hbm.at[idx_s.at[0]], rows_v.at[b],
                              sem_g.at[b]).wait()

    def start_s(b, chunk):
        pltpu.async_copy(rows_v.at[b], acc_sh.at[idx_d.at[chunk]],
                         sem_s.at[b], add=True)

    def wait_s(b):
        pltpu.make_async_copy(rows_v.at[b], acc_sh.at[idx_d.at[0]],
                              sem_s.at[b]).wait()

    for b in range(NBUF):
        start_g(b, b)

    def group(g, carry):
        first = g * NBUF
        for b in range(NBUF):
            wait_g(b)
            start_s(b, first + b)
        for b in range(NBUF):
            wait_s(b)
            start_g(b, first + b + NBUF)
        return carry

    lax.fori_loop(0, NGROUP - 1, group, 0)
    first = (NGROUP - 1) * NBUF
    for b in range(NBUF):
        wait_g(b)
        start_s(b, first + b)
    for b in range(NBUF):
        wait_s(b)
    plsc.subcore_barrier()
    pltpu.sync_copy(acc_sh.at[pl.ds(s * rows_per_tile, rows_per_tile)],
                    out_hbm.at[c].at[pl.ds(s * rows_per_tile, rows_per_tile)])


def _dis(degp_ref):
    deg = degp_ref[0, :N] + degp_ref[1, :N] + 1.0
    return lax.rsqrt(deg)


def _tc_pre_body(x_ref, lin_W_ref, lin_b_ref, W1_ref, degp_ref, ht_ref):
    dis = _dis(degp_ref)
    h0 = jnp.dot(x_ref[...], lin_W_ref[...], preferred_element_type=_F32)
    h0 = h0 + lin_b_ref[...][None, :]
    ht_ref[...] = dis[:, None] * jnp.dot(h0, W1_ref[...],
                                         preferred_element_type=_F32)


def _tc_mid_body(p_ref, ht_ref, degp_ref, b_ref, Wn_ref, out_ref):
    dis = _dis(degp_ref)
    acc = p_ref[0, :N, :] + p_ref[1, :N, :] + ht_ref[...]
    h = jnp.maximum(dis[:, None] * acc + b_ref[...][None, :], 0.0)
    out_ref[...] = dis[:, None] * jnp.dot(h, Wn_ref[...],
                                          preferred_element_type=_F32)


def _tc_post_body(p_ref, ht_ref, degp_ref, b3_ref, emb_W_ref, emb_b_ref,
                  batch_ref, pred_W_ref, pred_b_ref, out_ref):
    dis = _dis(degp_ref)
    acc = p_ref[0, :N, :] + p_ref[1, :N, :] + ht_ref[...]
    h = jnp.maximum(dis[:, None] * acc + b3_ref[...][None, :], 0.0)
    emb = jnp.dot(h, emb_W_ref[...], preferred_element_type=_F32)
    emb = jnp.maximum(emb + emb_b_ref[...][None, :], 0.0)
    gid = lax.broadcasted_iota(jnp.int32, (G, N), 0)
    onehot_t = (batch_ref[...][None, :] == gid).astype(_F32)
    pooled = jnp.dot(onehot_t, emb, preferred_element_type=_F32)
    w = pred_W_ref[...][:, 0][None, :]
    out_ref[...] = jnp.sum(pooled * w, axis=1) + pred_b_ref[0]


def _tc_call(body, out_shape, *args):
    return pl.pallas_call(body, out_shape=out_shape)(*args)


def kernel(x, edge_index, batch, lin_W, lin_b, W1, b1, W2, b2, W3, b3,
           emb_W, emb_b, pred_W, pred_b):
    src = edge_index[0]
    dst = edge_index[1]
    pad = E_PAD - E
    src_p = jnp.concatenate([src, jnp.zeros((pad,), jnp.int32)])
    dst_p = jnp.concatenate([dst, jnp.full((pad,), N, jnp.int32)])
    src_p = src_p.reshape(NW, NCHUNK, CH)
    dst_p = dst_p.reshape(NW, NCHUNK, CH)
    zeros1 = jnp.zeros((DEG_ROWS,), _F32)
    zeros2 = jnp.zeros((ACC_ROWS, H), _F32)

    degp = _sc_degree(dst_p, zeros1)

    ht1 = _tc_call(_tc_pre_body, jax.ShapeDtypeStruct((N, H), _F32),
                   x, lin_W, lin_b, W1, degp)
    p1 = _sc_aggregate(ht1, src_p, dst_p, zeros2)
    ht2 = _tc_call(_tc_mid_body, jax.ShapeDtypeStruct((N, H), _F32),
                   p1, ht1, degp, b1, W2)
    p2 = _sc_aggregate(ht2, src_p, dst_p, zeros2)
    ht3 = _tc_call(_tc_mid_body, jax.ShapeDtypeStruct((N, H), _F32),
                   p2, ht2, degp, b2, W3)
    p3 = _sc_aggregate(ht3, src_p, dst_p, zeros2)
    out = _tc_call(_tc_post_body, jax.ShapeDtypeStruct((G,), _F32),
                   p3, ht3, degp, b3, emb_W, emb_b, batch, pred_W, pred_b)
    return out
